# baseline (speedup 1.0000x reference)
import functools

import jax
import jax.numpy as jnp
from jax import lax
from jax.experimental import pallas as pl
from jax.experimental.pallas import tpu as pltpu
from jax.experimental.pallas import tpu_sc as plsc

NUM_EMBEDDINGS = 1000000
EMBEDDING_DIM = 64
BATCH = 16384
HIST = 20

TOTAL = BATCH * HIST
NUM_CORES = 2
NUM_SUBCORES = 16
NUM_WORKERS = NUM_CORES * NUM_SUBCORES
PER_WORKER = TOTAL // NUM_WORKERS
CHUNK = 512
NUM_CHUNKS = PER_WORKER // CHUNK

assert TOTAL % NUM_WORKERS == 0
assert PER_WORKER % CHUNK == 0


def _body(x_hbm, w_hbm, out_hbm, idx_v, rows_v, sem):
    wid = lax.axis_index("s") * NUM_CORES + lax.axis_index("c")
    base = wid * PER_WORKER

    def chunk_body(i, carry):
        off = base + i * CHUNK
        pltpu.sync_copy(x_hbm.at[pl.ds(off, CHUNK)], idx_v)
        pltpu.async_copy(w_hbm.at[idx_v], rows_v, sem).wait()
        pltpu.sync_copy(rows_v, out_hbm.at[pl.ds(off, CHUNK)])
        return carry

    lax.fori_loop(0, NUM_CHUNKS, chunk_body, 0)


@jax.jit
def kernel(x, weight):
    mesh = plsc.VectorSubcoreMesh(
        core_axis_name="c", subcore_axis_name="s",
        num_cores=NUM_CORES, num_subcores=NUM_SUBCORES,
    )
    flat_x = x.reshape(TOTAL)
    out = pl.kernel(
        _body,
        out_type=jax.ShapeDtypeStruct((TOTAL, EMBEDDING_DIM), jnp.float32),
        mesh=mesh,
        scratch_types=[
            pltpu.VMEM((CHUNK,), jnp.int32),
            pltpu.VMEM((CHUNK, EMBEDDING_DIM), jnp.float32),
            pltpu.SemaphoreType.DMA,
        ],
        compiler_params=pltpu.CompilerParams(use_tc_tiling_on_sc=False),
    )(flat_x, weight)
    return out.reshape(BATCH, HIST, EMBEDDING_DIM)


# speedup vs baseline: 1.0231x; 1.0231x over previous
import jax
import jax.numpy as jnp
from jax import lax
from jax.experimental import pallas as pl
from jax.experimental.pallas import tpu as pltpu
from jax.experimental.pallas import tpu_sc as plsc

NUM_EMBEDDINGS = 1000000
EMBEDDING_DIM = 64
BATCH = 16384
HIST = 20

TOTAL = BATCH * HIST
NUM_CORES = 2
NUM_SUBCORES = 16
NUM_WORKERS = NUM_CORES * NUM_SUBCORES
PER_WORKER = TOTAL // NUM_WORKERS
CHUNK = 512
NUM_CHUNKS = PER_WORKER // CHUNK
NBUF = 3

assert TOTAL % NUM_WORKERS == 0
assert PER_WORKER % CHUNK == 0


def _body(x_hbm, w_hbm, out_hbm, idx_v, rows_v, gsems, ssems):
    wid = lax.axis_index("s") * NUM_CORES + lax.axis_index("c")
    base = wid * PER_WORKER

    pltpu.sync_copy(x_hbm.at[pl.ds(base, PER_WORKER)], idx_v)

    def start_gather(c):
        b = c % NBUF
        return pltpu.async_copy(
            w_hbm.at[idx_v.at[pl.ds(c * CHUNK, CHUNK)]], rows_v.at[b],
            gsems[b])

    gd = {}
    for c in range(NBUF):
        gd[c] = start_gather(c)

    pending_stores = {}
    for c in range(NUM_CHUNKS):
        b = c % NBUF
        gd[c].wait()
        sd = pltpu.async_copy(
            rows_v.at[b], out_hbm.at[pl.ds(base + c * CHUNK, CHUNK)],
            ssems[b])
        if c + NBUF < NUM_CHUNKS:
            sd.wait()
            gd[c + NBUF] = start_gather(c + NBUF)
        else:
            pending_stores[b] = sd

    for b in sorted(pending_stores):
        pending_stores[b].wait()


@jax.jit
def kernel(x, weight):
    mesh = plsc.VectorSubcoreMesh(
        core_axis_name="c", subcore_axis_name="s",
        num_cores=NUM_CORES, num_subcores=NUM_SUBCORES,
    )
    flat_x = x.reshape(TOTAL)
    out = pl.kernel(
        _body,
        out_type=jax.ShapeDtypeStruct((TOTAL, EMBEDDING_DIM), jnp.float32),
        mesh=mesh,
        scratch_types=[
            pltpu.VMEM((PER_WORKER,), jnp.int32),
            pltpu.VMEM((NBUF, CHUNK, EMBEDDING_DIM), jnp.float32),
            [pltpu.SemaphoreType.DMA] * NBUF,
            [pltpu.SemaphoreType.DMA] * NBUF,
        ],
        compiler_params=pltpu.CompilerParams(use_tc_tiling_on_sc=False),
    )(flat_x, weight)
    return out.reshape(BATCH, HIST, EMBEDDING_DIM)


# speedup vs baseline: 1.0630x; 1.0390x over previous
import jax
import jax.numpy as jnp
from jax import lax
from jax.experimental import pallas as pl
from jax.experimental.pallas import tpu as pltpu
from jax.experimental.pallas import tpu_sc as plsc

NUM_EMBEDDINGS = 1000000
EMBEDDING_DIM = 64
BATCH = 16384
HIST = 20

TOTAL = BATCH * HIST
NUM_CORES = 2
NUM_SUBCORES = 16
NUM_WORKERS = NUM_CORES * NUM_SUBCORES
PER_WORKER = TOTAL // NUM_WORKERS
CHUNK = 256
NUM_CHUNKS = PER_WORKER // CHUNK
NBUF = 3

assert TOTAL % NUM_WORKERS == 0
assert PER_WORKER % CHUNK == 0


PAD_DIM = 128


def _body(x_hbm, w_hbm, out_hbm, idx_v, rows_v, gsems, ssems):
    wid = lax.axis_index("s") * NUM_CORES + lax.axis_index("c")
    base = wid * PER_WORKER

    pltpu.sync_copy(x_hbm.at[pl.ds(base, PER_WORKER)], idx_v)

    def start_gather(c):
        b = c % NBUF
        return pltpu.async_copy(
            w_hbm.at[idx_v.at[pl.ds(c * CHUNK, CHUNK)]], rows_v.at[b],
            gsems[b])

    gd = {}
    for c in range(NBUF):
        gd[c] = start_gather(c)

    pending_stores = {}
    for c in range(NUM_CHUNKS):
        b = c % NBUF
        gd[c].wait()
        sd = pltpu.async_copy(
            rows_v.at[b, :, pl.ds(0, EMBEDDING_DIM)],
            out_hbm.at[pl.ds(base + c * CHUNK, CHUNK)],
            ssems[b])
        if c + NBUF < NUM_CHUNKS:
            sd.wait()
            gd[c + NBUF] = start_gather(c + NBUF)
        else:
            pending_stores[b] = sd

    for b in sorted(pending_stores):
        pending_stores[b].wait()


@jax.jit
def kernel(x, weight):
    mesh = plsc.VectorSubcoreMesh(
        core_axis_name="c", subcore_axis_name="s",
        num_cores=NUM_CORES, num_subcores=NUM_SUBCORES,
    )
    flat_x = x.reshape(TOTAL)
    w_pad = jnp.pad(weight, ((0, 0), (0, PAD_DIM - EMBEDDING_DIM)))
    out = pl.kernel(
        _body,
        out_type=jax.ShapeDtypeStruct((TOTAL, EMBEDDING_DIM), jnp.float32),
        mesh=mesh,
        scratch_types=[
            pltpu.VMEM((PER_WORKER,), jnp.int32),
            pltpu.VMEM((NBUF, CHUNK, PAD_DIM), jnp.float32),
            [pltpu.SemaphoreType.DMA] * NBUF,
            [pltpu.SemaphoreType.DMA] * NBUF,
        ],
        compiler_params=pltpu.CompilerParams(use_tc_tiling_on_sc=False),
    )(flat_x, w_pad)
    return out.reshape(BATCH, HIST, EMBEDDING_DIM)
